# baseline (speedup 1.0000x reference)
import functools

import jax
import jax.numpy as jnp
from jax import lax
from jax.experimental import pallas as pl
from jax.experimental.pallas import tpu as pltpu
from jax.experimental.pallas import tpu_sc as plsc

D = 64
NC = 2
NS = 16
NW = NC * NS
CHUNK = 128


def _sc_gather_pairs(pairs, idx3):
    nw, nchunk, chunk = idx3.shape
    per_w = nchunk * chunk
    n = nw * per_w
    mesh = plsc.VectorSubcoreMesh(core_axis_name="c", subcore_axis_name="s")

    @functools.partial(
        pl.kernel,
        out_type=jax.ShapeDtypeStruct((n, 128), jnp.float32),
        mesh=mesh,
        scratch_types=[
            pltpu.VMEM((nchunk, chunk), jnp.int32),
            pltpu.VMEM((2, chunk, 128), jnp.float32),
            pltpu.SemaphoreType.DMA,
            pltpu.SemaphoreType.DMA,
            pltpu.SemaphoreType.DMA,
            pltpu.SemaphoreType.DMA,
        ],
    )
    def k(tab_hbm, idx_hbm, out_hbm, idx_v, buf, g_e, g_o, o_e, o_o):
        wid = lax.axis_index("s") * NC + lax.axis_index("c")
        base = wid * per_w
        pltpu.sync_copy(idx_hbm.at[wid], idx_v)

        def fire(i, par, sem):
            pltpu.async_copy(tab_hbm.at[idx_v.at[i]], buf.at[par], sem)

        def wait_gather(par, sem):
            pltpu.make_async_copy(
                tab_hbm.at[idx_v.at[0]], buf.at[par], sem).wait()

        def write_out(i, par, sem):
            pltpu.async_copy(
                buf.at[par], out_hbm.at[pl.ds(base + i * chunk, chunk)], sem)

        def wait_out(par, sem):
            pltpu.make_async_copy(
                buf.at[par], out_hbm.at[pl.ds(base, chunk)], sem).wait()

        fire(0, 0, g_e)

        @pl.loop(0, nchunk // 2)
        def _(p):
            i = 2 * p
            fire(i + 1, 1, g_o)
            wait_gather(0, g_e)
            write_out(i, 0, o_e)
            wait_out(0, o_e)

            @pl.when(i + 2 < nchunk)
            def _():
                fire(i + 2, 0, g_e)
            wait_gather(1, g_o)
            write_out(i + 1, 1, o_o)
            wait_out(1, o_o)

    return k(pairs, idx3)


def _tc_mlp_sel(x, par, W1, b1, W2, b2):
    n, _ = x.shape
    d = D
    blk = 4096

    def body(x_ref, p_ref, w1_ref, b1_ref, w2_ref, b2_ref, o_ref):
        xb = x_ref[...]
        sel = p_ref[...] != 0
        xs = jnp.where(sel, xb[:, d:], xb[:, :d])
        h = jnp.maximum(
            jnp.dot(xs, w1_ref[...], preferred_element_type=jnp.float32)
            + b1_ref[...], 0.0)
        o_ref[...] = (
            jnp.dot(h, w2_ref[...], preferred_element_type=jnp.float32)
            + b2_ref[...])

    return pl.pallas_call(
        body,
        grid=(n // blk,),
        in_specs=[
            pl.BlockSpec((blk, 2 * d), lambda i: (i, 0)),
            pl.BlockSpec((blk, 1), lambda i: (i, 0)),
            pl.BlockSpec((d, d), lambda i: (0, 0)),
            pl.BlockSpec((1, d), lambda i: (0, 0)),
            pl.BlockSpec((d, d), lambda i: (0, 0)),
            pl.BlockSpec((1, d), lambda i: (0, 0)),
        ],
        out_specs=pl.BlockSpec((blk, d), lambda i: (i, 0)),
        out_shape=jax.ShapeDtypeStruct((n, d), jnp.float32),
    )(x, par, W1, b1.reshape(1, d), W2, b2.reshape(1, d))


def kernel(class_indices, embedding, W1, b1, W2, b2):
    if class_indices.ndim == 1:
        class_indices = class_indices[:, None]
    q, b = class_indices.shape
    n = q * b
    per_w = n // NW
    nchunk = per_w // CHUNK
    V = embedding.shape[0]
    pairs = embedding.reshape(V // 2, 2 * D)
    idx = class_indices.reshape(-1).astype(jnp.int32)
    idx3 = (idx >> 1).reshape(NW, nchunk, CHUNK)
    par = (idx & 1).reshape(n, 1)
    gathered = _sc_gather_pairs(pairs, idx3)
    out = _tc_mlp_sel(gathered, par, W1, b1, W2, b2)
    return out.reshape(q, b, D)
